# baseline (device time: 180148 ns/iter reference)
import jax
import jax.numpy as jnp
from jax import lax
from jax.experimental import pallas as pl
from jax.experimental.pallas import tpu as pltpu

N_DEV = 4
SQ = 1024
SKV = 1024
HQ_LOCAL = 8
DH = 128
D_MODEL = 1024
D_HEADS_LOCAL = HQ_LOCAL * DH
SCALE = 0.08838834764831843


def _body(x_ref, wq_ref, k_ref, v_ref, wo_ref, out_ref,
          ctx_ref, comm_ref, send_sems, recv_sems):
    my = lax.axis_index("i")
    left = lax.rem(my - 1 + N_DEV, N_DEV)
    right = lax.rem(my + 1, N_DEV)

    barrier_sem = pltpu.get_barrier_semaphore()
    for nbr in (left, right):
        pl.semaphore_signal(
            barrier_sem, inc=1,
            device_id=(nbr,), device_id_type=pl.DeviceIdType.MESH,
        )
    pl.semaphore_wait(barrier_sem, 2)

    rows = lax.broadcasted_iota(jnp.int32, (SQ, SKV), 0)
    cols = lax.broadcasted_iota(jnp.int32, (SQ, SKV), 1)
    mask = ((rows // 64) % 4) == ((cols // 64) % 4)

    for h in range(HQ_LOCAL):
        q = jnp.dot(x_ref[:, :], wq_ref[:, h * DH:(h + 1) * DH],
                    preferred_element_type=jnp.float32) * SCALE
        s = lax.dot_general(q, k_ref[h], (((1,), (1,)), ((), ())),
                            preferred_element_type=jnp.float32)
        s = jnp.where(mask, s, -1e9)
        m = jnp.max(s, axis=1, keepdims=True)
        w = jnp.exp(s - m)
        w = w / jnp.sum(w, axis=1, keepdims=True)
        ctx_ref[:, h * DH:(h + 1) * DH] = jnp.dot(
            w, v_ref[h], preferred_element_type=jnp.float32)

    partial = jnp.dot(ctx_ref[:, :], wo_ref[:, :],
                      preferred_element_type=jnp.float32)
    out_ref[:, :] = partial
    comm_ref[0] = partial

    for h in range(N_DEV - 1):
        send_slot = h % 2
        recv_slot = (h + 1) % 2
        rdma = pltpu.make_async_remote_copy(
            src_ref=comm_ref.at[send_slot],
            dst_ref=comm_ref.at[recv_slot],
            send_sem=send_sems.at[send_slot],
            recv_sem=recv_sems.at[recv_slot],
            device_id=(right,),
            device_id_type=pl.DeviceIdType.MESH,
        )
        rdma.start()
        rdma.wait()
        out_ref[:, :] += comm_ref[recv_slot]


def kernel(x, Wq, K_ext, V_ext, Wo):
    p = lax.axis_index("i")
    x2 = x[0]
    k = jnp.transpose(K_ext[0], (1, 0, 2))
    v = jnp.transpose(V_ext[0], (1, 0, 2))
    wq_p = lax.dynamic_slice(Wq, (0, p * D_HEADS_LOCAL),
                             (D_MODEL, D_HEADS_LOCAL))
    wo_p = lax.dynamic_slice(Wo, (p * D_HEADS_LOCAL, 0),
                             (D_HEADS_LOCAL, D_MODEL))

    out = pl.pallas_call(
        _body,
        out_shape=jax.ShapeDtypeStruct((SQ, D_MODEL), jnp.float32),
        in_specs=[pl.BlockSpec(memory_space=pltpu.VMEM)] * 5,
        out_specs=pl.BlockSpec(memory_space=pltpu.VMEM),
        scratch_shapes=[
            pltpu.VMEM((SQ, D_HEADS_LOCAL), jnp.float32),
            pltpu.VMEM((2, SQ, D_MODEL), jnp.float32),
            pltpu.SemaphoreType.DMA((2,)),
            pltpu.SemaphoreType.DMA((2,)),
        ],
        compiler_params=pltpu.CompilerParams(collective_id=0),
    )(x2, wq_p, k, v, wo_p)
    return out[None]


# device time: 71396 ns/iter; 2.5232x vs baseline; 2.5232x over previous
import jax
import jax.numpy as jnp
from jax import lax
from jax.experimental import pallas as pl
from jax.experimental.pallas import tpu as pltpu

N_DEV = 4
SQ = 1024
HQ_LOCAL = 8
DH = 128
D_MODEL = 1024
D_HEADS_LOCAL = HQ_LOCAL * DH
SCALE = 0.08838834764831843
CHUNK = 128


def _orig_block_start(pb):
    return ((pb % 4) * 4 + pb // 4) * 64


def _store_chunk(out_ref, ring, c, val):
    base_pb = 2 * c + (8 if ring == 1 else 0)
    for j in range(2):
        pb = base_pb + j
        out_ref[pl.ds(_orig_block_start(pb), 64), :] = val[j * 64:(j + 1) * 64]


def _body(x_ref, wq_ref, k_ref, v_ref, wo_ref, out_ref,
          xp_ref, kp_ref, vp_ref, q_ref, ctx_ref, part_ref,
          commA, commB, semA_s, semA_r, semB_s, semB_r):
    my = lax.axis_index("i")
    left = lax.rem(my + N_DEV - 1, N_DEV)
    right = lax.rem(my + 1, N_DEV)

    barrier_sem = pltpu.get_barrier_semaphore()
    for nbr in (left, right):
        pl.semaphore_signal(
            barrier_sem, inc=1,
            device_id=(nbr,), device_id_type=pl.DeviceIdType.MESH,
        )
    pl.semaphore_wait(barrier_sem, 2)

    for pb in range(16):
        dst = pl.ds(pb * 64, 64)
        src = pl.ds(_orig_block_start(pb), 64)
        xp_ref[dst, :] = x_ref[src, :]
        kp_ref[dst, :] = k_ref[src, :]
        vp_ref[dst, :] = v_ref[src, :]

    q_ref[:, :] = jnp.dot(xp_ref[:, :], wq_ref[:, :],
                          preferred_element_type=jnp.float32) * SCALE

    for h in range(HQ_LOCAL):
        hc = slice(h * DH, (h + 1) * DH)
        for c in range(4):
            rs = slice(c * 256, (c + 1) * 256)
            qc = q_ref[rs, hc]
            kc = kp_ref[rs, hc]
            vc = vp_ref[rs, hc]
            s = lax.dot_general(qc, kc, (((1,), (1,)), ((), ())),
                                preferred_element_type=jnp.float32)
            m = jnp.max(s, axis=1, keepdims=True)
            w = jnp.exp(s - m)
            w = w / jnp.sum(w, axis=1, keepdims=True)
            ctx_ref[rs, hc] = jnp.dot(w, vc,
                                      preferred_element_type=jnp.float32)

    part_ref[:, :] = jnp.dot(ctx_ref[:, :], wo_ref[:, :],
                             preferred_element_type=jnp.float32)

    commA[0] = part_ref[pl.ds(my * CHUNK, CHUNK), :]
    commB[0] = part_ref[pl.ds(512 + my * CHUNK, CHUNK), :]

    for g in range(6):
        send_slot = g % 2
        recv_slot = (g + 1) % 2
        rdmaA = pltpu.make_async_remote_copy(
            src_ref=commA.at[send_slot], dst_ref=commA.at[recv_slot],
            send_sem=semA_s.at[send_slot], recv_sem=semA_r.at[recv_slot],
            device_id=(right,), device_id_type=pl.DeviceIdType.MESH,
        )
        rdmaB = pltpu.make_async_remote_copy(
            src_ref=commB.at[send_slot], dst_ref=commB.at[recv_slot],
            send_sem=semB_s.at[send_slot], recv_sem=semB_r.at[recv_slot],
            device_id=(left,), device_id_type=pl.DeviceIdType.MESH,
        )
        rdmaA.start()
        rdmaB.start()
        rdmaA.wait()
        rdmaB.wait()

        if g < 3:
            cA = lax.rem(my - g - 1 + 2 * N_DEV, N_DEV)
            cB = lax.rem(my + g + 1, N_DEV)
            commA[recv_slot] = commA[recv_slot] + part_ref[
                pl.ds(cA * CHUNK, CHUNK), :]
            commB[recv_slot] = commB[recv_slot] + part_ref[
                pl.ds(512 + cB * CHUNK, CHUNK), :]
            if g == 2:
                _store_chunk(out_ref, 0, lax.rem(my + 1, N_DEV),
                             commA[recv_slot])
                _store_chunk(out_ref, 1, lax.rem(my + N_DEV - 1, N_DEV),
                             commB[recv_slot])
        else:
            t = g - 3
            cA = lax.rem(my - t + N_DEV, N_DEV)
            cB = lax.rem(my + t, N_DEV)
            _store_chunk(out_ref, 0, cA, commA[recv_slot])
            _store_chunk(out_ref, 1, cB, commB[recv_slot])


def kernel(x, Wq, K_ext, V_ext, Wo):
    p = lax.axis_index("i")
    x2 = x[0]
    k2 = K_ext[0].reshape(SQ, D_HEADS_LOCAL)
    v2 = V_ext[0].reshape(SQ, D_HEADS_LOCAL)
    wq_p = lax.dynamic_slice(Wq, (0, p * D_HEADS_LOCAL),
                             (D_MODEL, D_HEADS_LOCAL))
    wo_p = lax.dynamic_slice(Wo, (p * D_HEADS_LOCAL, 0),
                             (D_HEADS_LOCAL, D_MODEL))

    out = pl.pallas_call(
        _body,
        out_shape=jax.ShapeDtypeStruct((SQ, D_MODEL), jnp.float32),
        in_specs=[pl.BlockSpec(memory_space=pltpu.VMEM)] * 5,
        out_specs=pl.BlockSpec(memory_space=pltpu.VMEM),
        scratch_shapes=[
            pltpu.VMEM((SQ, D_MODEL), jnp.float32),
            pltpu.VMEM((SQ, D_HEADS_LOCAL), jnp.float32),
            pltpu.VMEM((SQ, D_HEADS_LOCAL), jnp.float32),
            pltpu.VMEM((SQ, D_HEADS_LOCAL), jnp.float32),
            pltpu.VMEM((SQ, D_HEADS_LOCAL), jnp.float32),
            pltpu.VMEM((SQ, D_MODEL), jnp.float32),
            pltpu.VMEM((2, CHUNK, D_MODEL), jnp.float32),
            pltpu.VMEM((2, CHUNK, D_MODEL), jnp.float32),
            pltpu.SemaphoreType.DMA((2,)),
            pltpu.SemaphoreType.DMA((2,)),
            pltpu.SemaphoreType.DMA((2,)),
            pltpu.SemaphoreType.DMA((2,)),
        ],
        compiler_params=pltpu.CompilerParams(collective_id=0),
    )(x2, wq_p, k2, v2, wo_p)
    return out[None]


# device time: 29451 ns/iter; 6.1169x vs baseline; 2.4242x over previous
import jax
import jax.numpy as jnp
from jax import lax
from jax.experimental import pallas as pl
from jax.experimental.pallas import tpu as pltpu

N_DEV = 4
SQ = 1024
HQ_LOCAL = 8
DH = 128
D_MODEL = 1024
D_HEADS_LOCAL = HQ_LOCAL * DH
SCALE = 0.08838834764831843
CHUNK = 128


def _store_chunk(out_ref, ring, c, val):
    base_pb = 4 * c + (2 if ring == 1 else 0)
    for j in range(2):
        pb = base_pb + j
        orig = ((pb % 4) * 4 + pb // 4) * 64
        out_ref[pl.ds(orig, 64), :] = val[j * 64:(j + 1) * 64]


def _body(x_ref, wq_ref, k_ref, v_ref, wo_ref, out_ref,
          xp_ref, kp_ref, vp_ref, ctx_ref, part_ref,
          commA, commB, semA_s, semA_r, semB_s, semB_r):
    my = lax.axis_index("i")
    left = lax.rem(my + N_DEV - 1, N_DEV)
    right = lax.rem(my + 1, N_DEV)

    barrier_sem = pltpu.get_barrier_semaphore()
    for nbr in (left, right):
        pl.semaphore_signal(
            barrier_sem, inc=1,
            device_id=(nbr,), device_id_type=pl.DeviceIdType.MESH,
        )
    pl.semaphore_wait(barrier_sem, 2)

    for pb in range(16):
        dst = pl.ds(pb * 64, 64)
        src = pl.ds(((pb % 4) * 4 + pb // 4) * 64, 64)
        xp_ref[dst, :] = x_ref[src, :]
        kp_ref[dst, :] = k_ref[src, :]
        vp_ref[dst, :] = v_ref[src, :]

    def compute_class(cls):
        rows = pl.ds(cls * 256, 256)
        qc = jnp.dot(xp_ref[rows, :], wq_ref[:, :],
                     preferred_element_type=jnp.float32) * SCALE
        for h in range(HQ_LOCAL):
            hc = slice(h * DH, (h + 1) * DH)
            kc = kp_ref[rows, hc]
            vc = vp_ref[rows, hc]
            s = lax.dot_general(qc[:, hc], kc, (((1,), (1,)), ((), ())),
                                preferred_element_type=jnp.float32)
            m = jnp.max(s, axis=1, keepdims=True)
            w = jnp.exp(s - m)
            w = w / jnp.sum(w, axis=1, keepdims=True)
            ctx_ref[rows, hc] = jnp.dot(w, vc,
                                        preferred_element_type=jnp.float32)
        part_ref[rows, :] = jnp.dot(ctx_ref[rows, :], wo_ref[:, :],
                                    preferred_element_type=jnp.float32)

    compute_class(my)
    commA[0] = part_ref[pl.ds(my * 256, CHUNK), :]
    commB[0] = part_ref[pl.ds(my * 256 + CHUNK, CHUNK), :]

    for g in range(6):
        send_slot = g % 2
        recv_slot = (g + 1) % 2
        rdmaA = pltpu.make_async_remote_copy(
            src_ref=commA.at[send_slot], dst_ref=commA.at[recv_slot],
            send_sem=semA_s.at[send_slot], recv_sem=semA_r.at[recv_slot],
            device_id=(right,), device_id_type=pl.DeviceIdType.MESH,
        )
        rdmaB = pltpu.make_async_remote_copy(
            src_ref=commB.at[send_slot], dst_ref=commB.at[recv_slot],
            send_sem=semB_s.at[send_slot], recv_sem=semB_r.at[recv_slot],
            device_id=(left,), device_id_type=pl.DeviceIdType.MESH,
        )
        rdmaA.start()
        rdmaB.start()

        if g == 0:
            compute_class(lax.rem(my + 1, N_DEV))
            compute_class(lax.rem(my + 3, N_DEV))
        elif g == 1:
            compute_class(lax.rem(my + 2, N_DEV))

        rdmaA.wait()
        rdmaB.wait()

        if g < 3:
            cA = lax.rem(my - g - 1 + 2 * N_DEV, N_DEV)
            cB = lax.rem(my + g + 1, N_DEV)
            commA[recv_slot] = commA[recv_slot] + part_ref[
                pl.ds(cA * 256, CHUNK), :]
            commB[recv_slot] = commB[recv_slot] + part_ref[
                pl.ds(cB * 256 + CHUNK, CHUNK), :]
            if g == 2:
                _store_chunk(out_ref, 0, lax.rem(my + 1, N_DEV),
                             commA[recv_slot])
                _store_chunk(out_ref, 1, lax.rem(my + N_DEV - 1, N_DEV),
                             commB[recv_slot])
        else:
            t = g - 3
            cA = lax.rem(my - t + N_DEV, N_DEV)
            cB = lax.rem(my + t, N_DEV)
            _store_chunk(out_ref, 0, cA, commA[recv_slot])
            _store_chunk(out_ref, 1, cB, commB[recv_slot])


def kernel(x, Wq, K_ext, V_ext, Wo):
    p = lax.axis_index("i")
    x2 = x[0]
    k2 = K_ext[0].reshape(SQ, D_HEADS_LOCAL)
    v2 = V_ext[0].reshape(SQ, D_HEADS_LOCAL)
    wq_p = lax.dynamic_slice(Wq, (0, p * D_HEADS_LOCAL),
                             (D_MODEL, D_HEADS_LOCAL))
    wo_p = lax.dynamic_slice(Wo, (p * D_HEADS_LOCAL, 0),
                             (D_HEADS_LOCAL, D_MODEL))

    out = pl.pallas_call(
        _body,
        out_shape=jax.ShapeDtypeStruct((SQ, D_MODEL), jnp.float32),
        in_specs=[pl.BlockSpec(memory_space=pltpu.VMEM)] * 5,
        out_specs=pl.BlockSpec(memory_space=pltpu.VMEM),
        scratch_shapes=[
            pltpu.VMEM((SQ, D_MODEL), jnp.float32),
            pltpu.VMEM((SQ, D_HEADS_LOCAL), jnp.float32),
            pltpu.VMEM((SQ, D_HEADS_LOCAL), jnp.float32),
            pltpu.VMEM((SQ, D_HEADS_LOCAL), jnp.float32),
            pltpu.VMEM((SQ, D_MODEL), jnp.float32),
            pltpu.VMEM((2, CHUNK, D_MODEL), jnp.float32),
            pltpu.VMEM((2, CHUNK, D_MODEL), jnp.float32),
            pltpu.SemaphoreType.DMA((2,)),
            pltpu.SemaphoreType.DMA((2,)),
            pltpu.SemaphoreType.DMA((2,)),
            pltpu.SemaphoreType.DMA((2,)),
        ],
        compiler_params=pltpu.CompilerParams(collective_id=0),
    )(x2, wq_p, k2, v2, wo_p)
    return out[None]
